# baseline (speedup 1.0000x reference)
import functools

import jax
import jax.numpy as jnp
from jax import lax
from jax.experimental import pallas as pl
from jax.experimental.pallas import tpu as pltpu
from jax.experimental.pallas import tpu_sc as plsc

BATCH = 16384
DIM = 64
NUM_WORKERS = 32
ROWS_PER_WORKER = BATCH // NUM_WORKERS
LANES = 16
DIM_CHUNKS = DIM // LANES
NUM_GROUPS = ROWS_PER_WORKER // LANES
NUM_REL = 1_000
IDX_CHUNK = 128


def _body(heads_hbm, rels_hbm, tails_hbm, er_hbm, ei_hbm, rr_hbm, ri_hbm,
          out_hbm,
          rel_r_s, rel_i_s,
          idx_h, idx_r, idx_t,
          hr_v, hi_v, tr_v, ti_v, rr_all, ri_all,
          tmp_v, out_v,
          sem_hr, sem_hi, sem_tr, sem_ti, sem_rel):
    wid = lax.axis_index("s") * 2 + lax.axis_index("c")
    sid = lax.axis_index("s")
    base = wid * ROWS_PER_WORKER

    ROWS_PER_STAGER = (NUM_REL + 15) // 16
    stage_copies = []
    for m in range(ROWS_PER_STAGER):
        row_no = jnp.minimum(sid * ROWS_PER_STAGER + m, NUM_REL - 1)
        stage_copies.append(
            pltpu.async_copy(rr_hbm.at[row_no], rel_r_s.at[row_no], sem_rel))
        stage_copies.append(
            pltpu.async_copy(ri_hbm.at[row_no], rel_i_s.at[row_no], sem_rel))
    for c in stage_copies:
        c.wait()

    pltpu.sync_copy(heads_hbm.at[pl.ds(base, ROWS_PER_WORKER)], idx_h)
    pltpu.sync_copy(rels_hbm.at[pl.ds(base, ROWS_PER_WORKER)], idx_r)
    pltpu.sync_copy(tails_hbm.at[pl.ds(base, ROWS_PER_WORKER)], idx_t)

    plsc.subcore_barrier()

    lane = lax.iota(jnp.int32, LANES)

    def group_body(g, carry):
        off = g * LANES
        vh = idx_h[pl.ds(off, LANES)]
        vt = idx_t[pl.ds(off, LANES)]
        vr = idx_r[pl.ds(off, LANES)]
        copies = []
        for j in range(LANES):
            h = vh[j]
            t = vt[j]
            r = vr[j]
            copies.append(pltpu.async_copy(er_hbm.at[h], hr_v.at[j], sem_hr))
            copies.append(pltpu.async_copy(ei_hbm.at[h], hi_v.at[j], sem_hi))
            copies.append(pltpu.async_copy(er_hbm.at[t], tr_v.at[j], sem_tr))
            copies.append(pltpu.async_copy(ei_hbm.at[t], ti_v.at[j], sem_ti))
            copies.append(pltpu.async_copy(rel_r_s.at[r], rr_all.at[j],
                                           sem_rel))
            copies.append(pltpu.async_copy(rel_i_s.at[r], ri_all.at[j],
                                           sem_rel))
        for c in copies:
            c.wait()

        for j in range(LANES):
            acc = jnp.zeros((LANES,), jnp.float32)
            for d in range(DIM_CHUNKS):
                sl = pl.ds(d * LANES, LANES)
                hr = hr_v[j, sl]
                hi = hi_v[j, sl]
                tr = tr_v[j, sl]
                ti = ti_v[j, sl]
                rr = rr_all[j, sl]
                ri = ri_all[j, sl]
                acc = (acc + tr * (hr * rr + hi * ri)
                       + ti * (hr * ri - hi * rr))
            tmp_v[j, :] = acc
        scores = jnp.zeros((LANES,), jnp.float32)
        for k in range(LANES):
            col = jnp.full((LANES,), k, jnp.int32)
            scores = scores + plsc.load_gather(tmp_v, [lane, col])
        out_v[pl.ds(off, LANES)] = scores
        return carry

    lax.fori_loop(0, NUM_GROUPS, group_body, 0)

    pltpu.sync_copy(out_v, out_hbm.at[pl.ds(base, ROWS_PER_WORKER)])


@jax.jit
def _complex_score(heads, relations, tails, entity_real, entity_imag,
                   relation_real, relation_imag):
    mesh = plsc.VectorSubcoreMesh(core_axis_name="c", subcore_axis_name="s")
    kern = pl.kernel(
        _body,
        out_type=jax.ShapeDtypeStruct((BATCH,), jnp.float32),
        mesh=mesh,
        compiler_params=pltpu.CompilerParams(needs_layout_passes=False),
        scratch_types=[
            pltpu.VMEM_SHARED((NUM_REL, DIM), jnp.float32),
            pltpu.VMEM_SHARED((NUM_REL, DIM), jnp.float32),
            pltpu.VMEM((ROWS_PER_WORKER,), jnp.int32),
            pltpu.VMEM((ROWS_PER_WORKER,), jnp.int32),
            pltpu.VMEM((ROWS_PER_WORKER,), jnp.int32),
            pltpu.VMEM((LANES, DIM), jnp.float32),
            pltpu.VMEM((LANES, DIM), jnp.float32),
            pltpu.VMEM((LANES, DIM), jnp.float32),
            pltpu.VMEM((LANES, DIM), jnp.float32),
            pltpu.VMEM((LANES, DIM), jnp.float32),
            pltpu.VMEM((LANES, DIM), jnp.float32),
            pltpu.VMEM((LANES, LANES), jnp.float32),
            pltpu.VMEM((ROWS_PER_WORKER,), jnp.float32),
            pltpu.SemaphoreType.DMA,
            pltpu.SemaphoreType.DMA,
            pltpu.SemaphoreType.DMA,
            pltpu.SemaphoreType.DMA,
            pltpu.SemaphoreType.DMA,
        ],
    )
    return kern(heads, relations, tails, entity_real, entity_imag,
                relation_real, relation_imag)


def kernel(heads, relations, tails, entity_real, entity_imag,
           relation_real, relation_imag):
    return _complex_score(heads.astype(jnp.int32), relations.astype(jnp.int32),
                          tails.astype(jnp.int32), entity_real, entity_imag,
                          relation_real, relation_imag)


# speedup vs baseline: 1.0441x; 1.0441x over previous
import functools

import jax
import jax.numpy as jnp
from jax import lax
from jax.experimental import pallas as pl
from jax.experimental.pallas import tpu as pltpu
from jax.experimental.pallas import tpu_sc as plsc

BATCH = 16384
DIM = 64
NUM_WORKERS = 32
ROWS_PER_WORKER = BATCH // NUM_WORKERS
LANES = 16
DIM_CHUNKS = DIM // LANES
NUM_GROUPS = ROWS_PER_WORKER // LANES


def _body(heads_hbm, rels_hbm, tails_hbm, er_hbm, ei_hbm, rr_hbm, ri_hbm,
          out_hbm,
          idx_h, idx_r, idx_t,
          buf, tmp_v, out_v,
          sem_hr, sem_hi, sem_tr, sem_ti, sem_rr, sem_ri):
    wid = lax.axis_index("s") * 2 + lax.axis_index("c")
    base = wid * ROWS_PER_WORKER

    pltpu.sync_copy(heads_hbm.at[pl.ds(base, ROWS_PER_WORKER)], idx_h)
    pltpu.sync_copy(rels_hbm.at[pl.ds(base, ROWS_PER_WORKER)], idx_r)
    pltpu.sync_copy(tails_hbm.at[pl.ds(base, ROWS_PER_WORKER)], idx_t)

    lane = lax.iota(jnp.int32, LANES)
    sems = (sem_hr, sem_hi, sem_tr, sem_ti, sem_rr, sem_ri)

    def issue(g, slot):
        off = g * LANES
        vh = idx_h[pl.ds(off, LANES)]
        vt = idx_t[pl.ds(off, LANES)]
        vr = idx_r[pl.ds(off, LANES)]
        copies = []
        for j in range(LANES):
            h = vh[j]
            t = vt[j]
            r = vr[j]
            for k, (table, idx) in enumerate(
                    ((er_hbm, h), (ei_hbm, h), (er_hbm, t),
                     (ei_hbm, t), (rr_hbm, r), (ri_hbm, r))):
                copies.append(pltpu.async_copy(
                    table.at[idx], buf.at[slot, k, j], sems[k]))
        return copies

    def compute(g, slot):
        off = g * LANES
        for j in range(LANES):
            acc = jnp.zeros((LANES,), jnp.float32)
            for d in range(DIM_CHUNKS):
                sl = pl.ds(d * LANES, LANES)
                hr = buf[slot, 0, j, sl]
                hi = buf[slot, 1, j, sl]
                tr = buf[slot, 2, j, sl]
                ti = buf[slot, 3, j, sl]
                rr = buf[slot, 4, j, sl]
                ri = buf[slot, 5, j, sl]
                acc = (acc + tr * (hr * rr + hi * ri)
                       + ti * (hr * ri - hi * rr))
            tmp_v[j, :] = acc
        scores = jnp.zeros((LANES,), jnp.float32)
        for k in range(LANES):
            col = jnp.full((LANES,), k, jnp.int32)
            scores = scores + plsc.load_gather(tmp_v, [lane, col])
        out_v[pl.ds(off, LANES)] = scores

    prev = issue(0, 0)

    def group_body(g, carry):
        slot = lax.rem(g, 2)
        nxt = issue(g + 1, 1 - slot)
        off = g * LANES
        vh = idx_h[pl.ds(off, LANES)]
        vt = idx_t[pl.ds(off, LANES)]
        vr = idx_r[pl.ds(off, LANES)]
        for j in range(LANES):
            h = vh[j]
            t = vt[j]
            r = vr[j]
            for k, (table, idx) in enumerate(
                    ((er_hbm, h), (ei_hbm, h), (er_hbm, t),
                     (ei_hbm, t), (rr_hbm, r), (ri_hbm, r))):
                pltpu.make_async_copy(
                    table.at[idx], buf.at[slot, k, j], sems[k]).wait()
        compute(g, slot)
        return carry

    lax.fori_loop(0, NUM_GROUPS - 1, group_body, 0)
    last = NUM_GROUPS - 1
    slot_last = (NUM_GROUPS - 1) % 2
    off = last * LANES
    vh = idx_h[pl.ds(off, LANES)]
    vt = idx_t[pl.ds(off, LANES)]
    vr = idx_r[pl.ds(off, LANES)]
    for j in range(LANES):
        h = vh[j]
        t = vt[j]
        r = vr[j]
        for k, (table, idx) in enumerate(
                ((er_hbm, h), (ei_hbm, h), (er_hbm, t),
                 (ei_hbm, t), (rr_hbm, r), (ri_hbm, r))):
            pltpu.make_async_copy(
                table.at[idx], buf.at[slot_last, k, j], sems[k]).wait()
    compute(last, slot_last)

    pltpu.sync_copy(out_v, out_hbm.at[pl.ds(base, ROWS_PER_WORKER)])


@jax.jit
def _complex_score(heads, relations, tails, entity_real, entity_imag,
                   relation_real, relation_imag):
    mesh = plsc.VectorSubcoreMesh(core_axis_name="c", subcore_axis_name="s")
    kern = pl.kernel(
        _body,
        out_type=jax.ShapeDtypeStruct((BATCH,), jnp.float32),
        mesh=mesh,
        compiler_params=pltpu.CompilerParams(needs_layout_passes=False),
        scratch_types=[
            pltpu.VMEM((ROWS_PER_WORKER,), jnp.int32),
            pltpu.VMEM((ROWS_PER_WORKER,), jnp.int32),
            pltpu.VMEM((ROWS_PER_WORKER,), jnp.int32),
            pltpu.VMEM((2, 6, LANES, DIM), jnp.float32),
            pltpu.VMEM((LANES, LANES), jnp.float32),
            pltpu.VMEM((ROWS_PER_WORKER,), jnp.float32),
            pltpu.SemaphoreType.DMA,
            pltpu.SemaphoreType.DMA,
            pltpu.SemaphoreType.DMA,
            pltpu.SemaphoreType.DMA,
            pltpu.SemaphoreType.DMA,
            pltpu.SemaphoreType.DMA,
        ],
    )
    return kern(heads, relations, tails, entity_real, entity_imag,
                relation_real, relation_imag)


def kernel(heads, relations, tails, entity_real, entity_imag,
           relation_real, relation_imag):
    return _complex_score(heads.astype(jnp.int32), relations.astype(jnp.int32),
                          tails.astype(jnp.int32), entity_real, entity_imag,
                          relation_real, relation_imag)
